# baseline (speedup 1.0000x reference)
import jax
import jax.numpy as jnp
import numpy as np
from jax.experimental import pallas as pl

B, S = 4096, 50
NUM_CHAMP, NUM_ITEM, NUM_TRAIT = 60, 60, 27
D_CHAMP, D_ITEM, D_TRAIT = 30, 10, 8
STATS = 31

_SEGS = (
    (0, NUM_CHAMP),
    (3, NUM_ITEM), (4, NUM_ITEM), (5, NUM_ITEM),
    (6, NUM_TRAIT), (7, NUM_TRAIT), (8, NUM_TRAIT), (9, NUM_TRAIT),
    (10, NUM_TRAIT), (11, NUM_TRAIT), (12, NUM_TRAIT),
    (1, 4),
    (2, 15),
)
K_OH = sum(w for _, w in _SEGS)
D_EMB = D_CHAMP + 3 * D_ITEM + 7 * D_TRAIT
D_MM = D_EMB + 4 + 15
D_OUT = D_MM + STATS
D_IN = 13 + STATS

_NB = 4096

_sel = np.zeros((K_OH, D_IN), np.float32)
_loc = np.zeros((K_OH, 1), np.float32)
_j = 0
for _col, _w in _SEGS:
    _sel[_j:_j + _w, _col] = 1.0
    _loc[_j:_j + _w, 0] = np.arange(_w, dtype=np.float32)
    _j += _w


def _build_table(champ_table, item_table, trait_table):
    w = jnp.zeros((K_OH, D_MM), jnp.float32)
    r, c = 0, 0
    w = w.at[r:r + NUM_CHAMP, c:c + D_CHAMP].set(champ_table)
    r += NUM_CHAMP
    c += D_CHAMP
    for _ in range(3):
        w = w.at[r:r + NUM_ITEM, c:c + D_ITEM].set(item_table)
        r += NUM_ITEM
        c += D_ITEM
    for _ in range(7):
        w = w.at[r:r + NUM_TRAIT, c:c + D_TRAIT].set(trait_table)
        r += NUM_TRAIT
        c += D_TRAIT
    w = w.at[r:r + 4, c:c + 4].set(jnp.eye(4, dtype=jnp.float32))
    r += 4
    c += 4
    w = w.at[r:r + 15, c:c + 15].set(jnp.eye(15, dtype=jnp.float32))
    return w.T.astype(jnp.bfloat16)


_SB = 5


def _body(x_ref, s_ref, l_ref, w_ref, o_ref):
    for s in range(_SB):
        xs = x_ref[s]
        f = jnp.dot(s_ref[...], xs.astype(jnp.bfloat16),
                    preferred_element_type=jnp.float32)
        onehot = (f == l_ref[...]).astype(jnp.bfloat16)
        emb = jnp.dot(w_ref[...], onehot,
                      preferred_element_type=jnp.float32)
        o_ref[s] = jnp.concatenate([emb, xs[13:, :]], axis=0)


@jax.jit
def kernel(x, champ_table, item_table, trait_table):
    w = _build_table(champ_table, item_table, trait_table)
    sel = jnp.asarray(_sel, dtype=jnp.bfloat16)
    loc = jnp.asarray(_loc)
    xt = jnp.transpose(x, (1, 2, 0))
    out_t = pl.pallas_call(
        _body,
        grid=(S // _SB, B // _NB),
        in_specs=[
            pl.BlockSpec((_SB, D_IN, _NB), lambda s, i: (s, 0, i)),
            pl.BlockSpec((K_OH, D_IN), lambda s, i: (0, 0)),
            pl.BlockSpec((K_OH, 1), lambda s, i: (0, 0)),
            pl.BlockSpec((D_MM, K_OH), lambda s, i: (0, 0)),
        ],
        out_specs=pl.BlockSpec((_SB, D_OUT, _NB), lambda s, i: (s, 0, i)),
        out_shape=jax.ShapeDtypeStruct((S, D_OUT, B), jnp.float32),
    )(xt, sel, loc, w)
    return jnp.transpose(out_t, (2, 0, 1))
